# Initial kernel scaffold; baseline (speedup 1.0000x reference)
import jax
import jax.numpy as jnp
from jax.experimental import pallas as pl


def kernel(x_in, W1, b1, W2, b2, W3, b3, W4, b4, Wr1, br1, Wcm2, bcm2, Wcm3, bcm3):
    raise NotImplementedError("write your pallas kernel here")


# speedup vs baseline: 2.2343x; 2.2343x over previous
import jax
import jax.numpy as jnp
from jax import lax
from jax.experimental import pallas as pl

_B, _CIN, _H, _W = 1, 128, 448, 64
_CLASSES, _SUPER = 256, 16
_CF = _CLASSES // _SUPER
_CL = 128
_R0, _R1 = 256, 8
_NH = 8
_GRID = _H // _NH


def _leaky(x):
    return jnp.where(x >= 0, x, 0.01 * x)


def _dotT(a, b):
    return lax.dot_general(a, b, (((1,), (1,)), ((), ())),
                           preferred_element_type=jnp.float32)


def _dot(a, b):
    return lax.dot_general(a, b, (((1,), (0,)), ((), ())),
                           preferred_element_type=jnp.float32)


def _kern(x_ref, w1_ref, b1_ref, w2_ref, b2_ref, w3_ref, b3_ref,
          w4_ref, b4_ref, wr1_ref, br1_ref, wcm2_ref, bcm2_ref,
          wcm3_ref, xreal_ref, mask_ref):
    for j in range(_NH):
        x = x_ref[j]
        xr = _leaky(_dotT(x, wr1_ref[j]) + br1_ref[j])
        h1 = _leaky(_dotT(x, w1_ref[j]) + b1_ref[j])
        h2 = _leaky(_dotT(h1, w2_ref[j]) + b2_ref[j])
        h3 = _leaky(_dotT(h2, w3_ref[j]) + b3_ref[j])
        y = _dotT(h3, w4_ref[j]) + b4_ref[j]
        ycls = y[:, :_CLASSES]
        cio = lax.broadcasted_iota(jnp.int32, (_W, _CLASSES), 1)
        mx = jnp.max(ycls, axis=1, keepdims=True)
        ind = jnp.min(jnp.where(ycls == mx, cio, _CLASSES),
                      axis=1, keepdims=True)
        zall = _leaky(_dot(xr, wcm2_ref[j]) + bcm2_ref[j])
        c2 = lax.broadcasted_iota(jnp.int32, (_W, _SUPER * _R1), 1)
        zm = jnp.where((c2 // _R1) == (ind // _CF), zall, 0.0)
        fold = (lax.broadcasted_iota(jnp.int32, (_SUPER * _R1, _R1), 0) % _R1
                == lax.broadcasted_iota(jnp.int32, (_SUPER * _R1, _R1), 1)
                ).astype(jnp.float32)
        zsel = _dot(zm, fold)
        onehot = (cio == ind).astype(jnp.float32)
        g3 = _dot(onehot, wcm3_ref[j])
        r = (jnp.sum(zsel * g3[:, :_R1], axis=1, keepdims=True)
             + g3[:, _R1:_R1 + 1])
        xreal_ref[j] = (ind.astype(jnp.float32) + r) * (1.0 / _CLASSES)
        mask_ref[j] = _leaky(y[:, _CLASSES:_CLASSES + 1])


def kernel(x_in, W1, b1, W2, b2, W3, b3, W4, b4, Wr1, br1, Wcm2, bcm2,
           Wcm3, bcm3):
    xt = jnp.transpose(x_in[0], (1, 2, 0))
    wcm2t = jnp.transpose(Wcm2.reshape(_H, _SUPER, _R0, _R1),
                          (0, 2, 1, 3)).reshape(_H, _R0, _SUPER * _R1)
    bcm2r = bcm2.reshape(_H, 1, _SUPER * _R1)
    wcm3aug = jnp.concatenate(
        [Wcm3.reshape(_H, _CLASSES, _R1), bcm3.reshape(_H, _CLASSES, 1)],
        axis=2)

    def im(i):
        return (i, 0, 0)

    spec = lambda s: pl.BlockSpec(s, im)
    xreal, mask = pl.pallas_call(
        _kern,
        grid=(_GRID,),
        in_specs=[
            spec((_NH, _W, _CIN)),
            spec((_NH, _CL, _CIN)), spec((_NH, 1, _CL)),
            spec((_NH, _CL, _CL)), spec((_NH, 1, _CL)),
            spec((_NH, _CL, _CL)), spec((_NH, 1, _CL)),
            spec((_NH, _CLASSES + 1, _CL)), spec((_NH, 1, _CLASSES + 1)),
            spec((_NH, _R0, _CIN)), spec((_NH, 1, _R0)),
            spec((_NH, _R0, _SUPER * _R1)), spec((_NH, 1, _SUPER * _R1)),
            spec((_NH, _CLASSES, _R1 + 1)),
        ],
        out_specs=[spec((_NH, _W, 1)), spec((_NH, _W, 1))],
        out_shape=[
            jax.ShapeDtypeStruct((_H, _W, 1), jnp.float32),
            jax.ShapeDtypeStruct((_H, _W, 1), jnp.float32),
        ],
    )(xt, W1, b1[:, None, :], W2, b2[:, None, :], W3, b3[:, None, :],
      W4, b4[:, None, :], Wr1, br1[:, None, :], wcm2t, bcm2r, wcm3aug)

    return (xreal.reshape(1, 1, _H, _W), mask.reshape(1, 1, _H, _W))


# speedup vs baseline: 3.9096x; 1.7498x over previous
import jax
import jax.numpy as jnp
from jax import lax
from jax.experimental import pallas as pl

_B, _CIN, _H, _W = 1, 128, 448, 64
_CLASSES, _SUPER = 256, 16
_CF = _CLASSES // _SUPER
_CL = 128
_R0, _R1 = 256, 8
_NH = 8
_GRID = _H // _NH


def _leaky(x):
    return jnp.where(x >= 0, x, 0.01 * x)


def _dotT(a, b):
    return lax.dot_general(a, b, (((1,), (1,)), ((), ())),
                           preferred_element_type=jnp.float32)


def _dot(a, b):
    return lax.dot_general(a, b, (((1,), (0,)), ((), ())),
                           preferred_element_type=jnp.float32)


def _kern(x_ref, w1_ref, b1_ref, w2_ref, b2_ref, w3_ref, b3_ref,
          w4_ref, b4_ref, wr1_ref, br1_ref, wcm2_ref, bcm2_ref,
          wcm3_ref, xreal_ref, mask_ref):
    rng = range(_NH)
    xs = [x_ref[j] for j in rng]
    xrs = [_leaky(_dotT(xs[j], wr1_ref[j]) + br1_ref[j]) for j in rng]
    h1 = [_leaky(_dotT(xs[j], w1_ref[j]) + b1_ref[j]) for j in rng]
    h2 = [_leaky(_dotT(h1[j], w2_ref[j]) + b2_ref[j]) for j in rng]
    h3 = [_leaky(_dotT(h2[j], w3_ref[j]) + b3_ref[j]) for j in rng]
    ys = [_dotT(h3[j], w4_ref[j]) + b4_ref[j] for j in rng]
    zalls = [_leaky(_dot(xrs[j], wcm2_ref[j]) + bcm2_ref[j]) for j in rng]
    cio = lax.broadcasted_iota(jnp.int32, (_W, _CLASSES), 1)
    c2 = lax.broadcasted_iota(jnp.int32, (_W, _SUPER * _R1), 1)
    fold = (lax.broadcasted_iota(jnp.int32, (_SUPER * _R1, _R1), 0) % _R1
            == lax.broadcasted_iota(jnp.int32, (_SUPER * _R1, _R1), 1)
            ).astype(jnp.float32)
    for j in rng:
        y = ys[j]
        ycls = y[:, :_CLASSES]
        mx = jnp.max(ycls, axis=1, keepdims=True)
        ind = jnp.min(jnp.where(ycls == mx, cio, _CLASSES),
                      axis=1, keepdims=True)
        zm = jnp.where((c2 // _R1) == (ind // _CF), zalls[j], 0.0)
        zsel = _dot(zm, fold)
        onehot = (cio == ind).astype(jnp.float32)
        g3 = _dot(onehot, wcm3_ref[j])
        r = (jnp.sum(zsel * g3[:, :_R1], axis=1, keepdims=True)
             + g3[:, _R1:_R1 + 1])
        xreal_ref[j] = (ind.astype(jnp.float32) + r) * (1.0 / _CLASSES)
        mask_ref[j] = _leaky(y[:, _CLASSES:_CLASSES + 1])


def kernel(x_in, W1, b1, W2, b2, W3, b3, W4, b4, Wr1, br1, Wcm2, bcm2,
           Wcm3, bcm3):
    xt = jnp.transpose(x_in[0], (1, 2, 0))
    wcm2t = jnp.transpose(Wcm2.reshape(_H, _SUPER, _R0, _R1),
                          (0, 2, 1, 3)).reshape(_H, _R0, _SUPER * _R1)
    bcm2r = bcm2.reshape(_H, 1, _SUPER * _R1)
    wcm3aug = jnp.concatenate(
        [Wcm3.reshape(_H, _CLASSES, _R1), bcm3.reshape(_H, _CLASSES, 1)],
        axis=2)

    def im(i):
        return (i, 0, 0)

    spec = lambda s: pl.BlockSpec(s, im)
    xreal, mask = pl.pallas_call(
        _kern,
        grid=(_GRID,),
        in_specs=[
            spec((_NH, _W, _CIN)),
            spec((_NH, _CL, _CIN)), spec((_NH, 1, _CL)),
            spec((_NH, _CL, _CL)), spec((_NH, 1, _CL)),
            spec((_NH, _CL, _CL)), spec((_NH, 1, _CL)),
            spec((_NH, _CLASSES + 1, _CL)), spec((_NH, 1, _CLASSES + 1)),
            spec((_NH, _R0, _CIN)), spec((_NH, 1, _R0)),
            spec((_NH, _R0, _SUPER * _R1)), spec((_NH, 1, _SUPER * _R1)),
            spec((_NH, _CLASSES, _R1 + 1)),
        ],
        out_specs=[spec((_NH, _W, 1)), spec((_NH, _W, 1))],
        out_shape=[
            jax.ShapeDtypeStruct((_H, _W, 1), jnp.float32),
            jax.ShapeDtypeStruct((_H, _W, 1), jnp.float32),
        ],
    )(xt, W1, b1[:, None, :], W2, b2[:, None, :], W3, b3[:, None, :],
      W4, b4[:, None, :], Wr1, br1[:, None, :], wcm2t, bcm2r, wcm3aug)

    return (xreal.reshape(1, 1, _H, _W), mask.reshape(1, 1, _H, _W))


# speedup vs baseline: 4.2046x; 1.0754x over previous
import jax
import jax.numpy as jnp
from jax import lax
from jax.experimental import pallas as pl

_B, _CIN, _H, _W = 1, 128, 448, 64
_CLASSES, _SUPER = 256, 16
_CF = _CLASSES // _SUPER
_CL = 128
_R0, _R1 = 256, 8
_NH = 8
_GRID = _H // _NH


def _leaky(x):
    return jnp.where(x >= 0, x, 0.01 * x)


def _dotT(a, b):
    return lax.dot_general(a, b, (((1,), (1,)), ((), ())),
                           preferred_element_type=jnp.float32)


def _dot(a, b):
    return lax.dot_general(a, b, (((1,), (0,)), ((), ())),
                           preferred_element_type=jnp.float32)


def _kern(x_ref, w1_ref, b1_ref, w2_ref, b2_ref, w3_ref, b3_ref,
          w4_ref, b4_ref, wr1_ref, br1_ref, wcm2_ref, bcm2_ref,
          wcm3_ref, bcm3_ref, xreal_ref, mask_ref):
    rng = range(_NH)
    xs = [x_ref[:, j, :].T for j in rng]
    xrs = [_leaky(_dotT(xs[j], wr1_ref[j]) + br1_ref[j]) for j in rng]
    h1 = [_leaky(_dotT(xs[j], w1_ref[j]) + b1_ref[j]) for j in rng]
    h2 = [_leaky(_dotT(h1[j], w2_ref[j]) + b2_ref[j]) for j in rng]
    h3 = [_leaky(_dotT(h2[j], w3_ref[j]) + b3_ref[j]) for j in rng]
    ys = [_dotT(h3[j], w4_ref[j]) + b4_ref[j] for j in rng]
    zalls = [_leaky(_dot(xrs[j], wcm2_ref[j]) + bcm2_ref[j]) for j in rng]
    cio = lax.broadcasted_iota(jnp.int32, (_W, _CLASSES), 1)
    c2 = lax.broadcasted_iota(jnp.int32, (_W, _SUPER * _R1), 1)
    fold = (lax.broadcasted_iota(jnp.int32, (_SUPER * _R1, _R1), 0) % _R1
            == lax.broadcasted_iota(jnp.int32, (_SUPER * _R1, _R1), 1)
            ).astype(jnp.float32)
    for j in rng:
        y = ys[j]
        ycls = y[:, :_CLASSES]
        mx = jnp.max(ycls, axis=1, keepdims=True)
        ind = jnp.min(jnp.where(ycls == mx, cio, _CLASSES),
                      axis=1, keepdims=True)
        zm = jnp.where((c2 // _R1) == (ind // _CF), zalls[j], 0.0)
        zsel = _dot(zm, fold)
        onehot = (cio == ind).astype(jnp.float32)
        g3 = _dotT(onehot, wcm3_ref[j])
        bsel = jnp.sum(onehot * bcm3_ref[j], axis=1, keepdims=True)
        r = jnp.sum(zsel * g3, axis=1, keepdims=True) + bsel
        xreal_ref[j] = (ind.astype(jnp.float32) + r) * (1.0 / _CLASSES)
        mask_ref[j] = _leaky(y[:, _CLASSES:_CLASSES + 1])


def kernel(x_in, W1, b1, W2, b2, W3, b3, W4, b4, Wr1, br1, Wcm2, bcm2,
           Wcm3, bcm3):
    xn = x_in.reshape(_CIN, _H, _W)
    wcm2t = jnp.transpose(Wcm2.reshape(_H, _SUPER, _R0, _R1),
                          (0, 2, 1, 3)).reshape(_H, _R0, _SUPER * _R1)
    bcm2r = bcm2.reshape(_H, 1, _SUPER * _R1)
    wcm3t = jnp.transpose(Wcm3.reshape(_H, _CLASSES, _R1), (0, 2, 1))
    bcm3r = bcm3.reshape(_H, 1, _CLASSES)

    def im(i):
        return (i, 0, 0)

    spec = lambda s: pl.BlockSpec(s, im)
    xreal, mask = pl.pallas_call(
        _kern,
        grid=(_GRID,),
        in_specs=[
            pl.BlockSpec((_CIN, _NH, _W), lambda i: (0, i, 0)),
            spec((_NH, _CL, _CIN)), spec((_NH, 1, _CL)),
            spec((_NH, _CL, _CL)), spec((_NH, 1, _CL)),
            spec((_NH, _CL, _CL)), spec((_NH, 1, _CL)),
            spec((_NH, _CLASSES + 1, _CL)), spec((_NH, 1, _CLASSES + 1)),
            spec((_NH, _R0, _CIN)), spec((_NH, 1, _R0)),
            spec((_NH, _R0, _SUPER * _R1)), spec((_NH, 1, _SUPER * _R1)),
            spec((_NH, _R1, _CLASSES)), spec((_NH, 1, _CLASSES)),
        ],
        out_specs=[spec((_NH, _W, 1)), spec((_NH, _W, 1))],
        out_shape=[
            jax.ShapeDtypeStruct((_H, _W, 1), jnp.float32),
            jax.ShapeDtypeStruct((_H, _W, 1), jnp.float32),
        ],
    )(xn, W1, b1[:, None, :], W2, b2[:, None, :], W3, b3[:, None, :],
      W4, b4[:, None, :], Wr1, br1[:, None, :], wcm2t, bcm2r, wcm3t, bcm3r)

    return (xreal.reshape(1, 1, _H, _W), mask.reshape(1, 1, _H, _W))


# speedup vs baseline: 4.4843x; 1.0665x over previous
import jax
import jax.numpy as jnp
from jax import lax
from jax.experimental import pallas as pl

_B, _CIN, _H, _W = 1, 128, 448, 64
_CLASSES, _SUPER = 256, 16
_CF = _CLASSES // _SUPER
_CL = 128
_R0, _R1 = 256, 8
_NH = 16
_GRID = _H // _NH


def _leaky(x):
    return jnp.where(x >= 0, x, 0.01 * x)


def _dotT(a, b):
    return lax.dot_general(a, b, (((1,), (1,)), ((), ())),
                           preferred_element_type=jnp.float32)


def _dot(a, b):
    return lax.dot_general(a, b, (((1,), (0,)), ((), ())),
                           preferred_element_type=jnp.float32)


def _kern(x_ref, w1_ref, b1_ref, w2_ref, b2_ref, w3_ref, b3_ref,
          w4_ref, b4_ref, wr1_ref, br1_ref, wcm2_ref, bcm2_ref,
          wcm3_ref, bcm3_ref, xreal_ref, mask_ref):
    rng = range(_NH)
    xs = [x_ref[:, j, :].T for j in rng]
    xrs = [_leaky(_dotT(xs[j], wr1_ref[j]) + br1_ref[j]) for j in rng]
    h1 = [_leaky(_dotT(xs[j], w1_ref[j]) + b1_ref[j]) for j in rng]
    h2 = [_leaky(_dotT(h1[j], w2_ref[j]) + b2_ref[j]) for j in rng]
    h3 = [_leaky(_dotT(h2[j], w3_ref[j]) + b3_ref[j]) for j in rng]
    ys = [_dotT(h3[j], w4_ref[j]) + b4_ref[j] for j in rng]
    zalls = [_leaky(_dot(xrs[j], wcm2_ref[j]) + bcm2_ref[j]) for j in rng]
    cio = lax.broadcasted_iota(jnp.int32, (_W, _CLASSES), 1)
    c2 = lax.broadcasted_iota(jnp.int32, (_W, _SUPER * _R1), 1)
    fold = (lax.broadcasted_iota(jnp.int32, (_SUPER * _R1, _R1), 0) % _R1
            == lax.broadcasted_iota(jnp.int32, (_SUPER * _R1, _R1), 1)
            ).astype(jnp.float32)
    for j in rng:
        y = ys[j]
        ycls = y[:, :_CLASSES]
        mx = jnp.max(ycls, axis=1, keepdims=True)
        ind = jnp.min(jnp.where(ycls == mx, cio, _CLASSES),
                      axis=1, keepdims=True)
        zm = jnp.where((c2 // _R1) == (ind // _CF), zalls[j], 0.0)
        zsel = _dot(zm, fold)
        onehot = (cio == ind).astype(jnp.float32)
        g3 = _dotT(onehot, wcm3_ref[j])
        bsel = jnp.sum(onehot * bcm3_ref[j], axis=1, keepdims=True)
        r = jnp.sum(zsel * g3, axis=1, keepdims=True) + bsel
        xreal_ref[j] = (ind.astype(jnp.float32) + r) * (1.0 / _CLASSES)
        mask_ref[j] = _leaky(y[:, _CLASSES:_CLASSES + 1])


def kernel(x_in, W1, b1, W2, b2, W3, b3, W4, b4, Wr1, br1, Wcm2, bcm2,
           Wcm3, bcm3):
    xn = x_in.reshape(_CIN, _H, _W)
    wcm2t = jnp.transpose(Wcm2.reshape(_H, _SUPER, _R0, _R1),
                          (0, 2, 1, 3)).reshape(_H, _R0, _SUPER * _R1)
    bcm2r = bcm2.reshape(_H, 1, _SUPER * _R1)
    wcm3t = jnp.transpose(Wcm3.reshape(_H, _CLASSES, _R1), (0, 2, 1))
    bcm3r = bcm3.reshape(_H, 1, _CLASSES)

    def im(i):
        return (i, 0, 0)

    spec = lambda s: pl.BlockSpec(s, im)
    xreal, mask = pl.pallas_call(
        _kern,
        grid=(_GRID,),
        in_specs=[
            pl.BlockSpec((_CIN, _NH, _W), lambda i: (0, i, 0)),
            spec((_NH, _CL, _CIN)), spec((_NH, 1, _CL)),
            spec((_NH, _CL, _CL)), spec((_NH, 1, _CL)),
            spec((_NH, _CL, _CL)), spec((_NH, 1, _CL)),
            spec((_NH, _CLASSES + 1, _CL)), spec((_NH, 1, _CLASSES + 1)),
            spec((_NH, _R0, _CIN)), spec((_NH, 1, _R0)),
            spec((_NH, _R0, _SUPER * _R1)), spec((_NH, 1, _SUPER * _R1)),
            spec((_NH, _R1, _CLASSES)), spec((_NH, 1, _CLASSES)),
        ],
        out_specs=[spec((_NH, _W, 1)), spec((_NH, _W, 1))],
        out_shape=[
            jax.ShapeDtypeStruct((_H, _W, 1), jnp.float32),
            jax.ShapeDtypeStruct((_H, _W, 1), jnp.float32),
        ],
    )(xn, W1, b1[:, None, :], W2, b2[:, None, :], W3, b3[:, None, :],
      W4, b4[:, None, :], Wr1, br1[:, None, :], wcm2t, bcm2r, wcm3t, bcm3r)

    return (xreal.reshape(1, 1, _H, _W), mask.reshape(1, 1, _H, _W))
